# baseline (speedup 1.0000x reference)
import jax
import jax.numpy as jnp
from jax.experimental import pallas as pl

N_UP = 16
N_DOWN = 16
N_ELEC = 32
N_ATOMS = 16
N_NBR = N_ELEC + N_ATOMS
BASIS = 4
KER = 8
EMBED = 16
LAYERS = 2
BATCH = 4096
LANES = N_NBR * KER

B_BLK = 128


def _jastrow_kernel(xs_ref, wbig_ref, bbig_ref, mask_ref, diag_ref, sel_ref,
                    nuc_ref, wht_ref, bht_ref, wg_ref, bg_ref, wo_ref, x0_ref,
                    bo_ref, out_ref):
    B = B_BLK
    R = B * N_ELEC
    X = xs_ref[...]

    x = jnp.broadcast_to(x0_ref[...][None], (B, N_ELEC, EMBED)).reshape(R, EMBED)

    mask = mask_ref[...]
    diag = diag_ref[...]
    nuc_row = nuc_ref[...]

    for l in range(LAYERS):
        P = jnp.tanh(
            jax.lax.dot(X, wbig_ref[l], preferred_element_type=jnp.float32)
            + bbig_ref[l][None, :])
        P = (P.reshape(B, N_ELEC, LANES) * mask[None]).reshape(R, LANES)

        pre_h = (jax.lax.dot(x, wht_ref[l], preferred_element_type=jnp.float32)
                 + bht_ref[l][None, :])
        hb = jnp.tanh((pre_h.reshape(B, N_ELEC, LANES) * diag[None]).sum(axis=1))
        Hbig = hb + nuc_row

        M = (P.reshape(B, N_ELEC, LANES) * Hbig[:, None, :]).reshape(R, LANES)
        z = jax.lax.dot(M, sel_ref[...], preferred_element_type=jnp.float32)

        x = x + jnp.tanh(
            jax.lax.dot(z, wg_ref[l], preferred_element_type=jnp.float32)
            + bg_ref[l][None, :])

    t = (x * wo_ref[...]).sum(axis=1, keepdims=True)
    out_ref[...] = (t.reshape(B, N_ELEC, 1).sum(axis=1)
                    + jnp.float32(N_ELEC) * bo_ref[0, 0])


@jax.jit
def kernel(xs, elec_embed, nuc_embed, Ww_e, bw_e, Ww_n, bw_n, Wh, bh, Wg, bg,
           Wo, bo):
    f32 = jnp.float32
    xs_flat = xs.reshape(BATCH * N_ELEC, N_NBR * BASIS)

    eye48 = jnp.eye(N_NBR, dtype=f32)
    w_sel = jnp.concatenate(
        [jnp.broadcast_to(Ww_e[:, None], (LAYERS, N_ELEC, BASIS, KER)),
         jnp.broadcast_to(Ww_n[:, None], (LAYERS, N_ATOMS, BASIS, KER))],
        axis=1)
    w_big = jnp.einsum('jJ,ljfk->ljfJk', eye48, w_sel).reshape(
        LAYERS, N_NBR * BASIS, LANES)
    b_big = jnp.concatenate(
        [jnp.broadcast_to(bw_e[:, None], (LAYERS, N_ELEC, KER)),
         jnp.broadcast_to(bw_n[:, None], (LAYERS, N_ATOMS, KER))],
        axis=1).reshape(LAYERS, LANES)

    eye_jk = jnp.repeat(jnp.eye(N_ELEC, N_NBR, dtype=f32), KER, axis=1)
    mask = 1.0 - eye_jk
    diag = eye_jk

    sel = jnp.tile(jnp.eye(KER, dtype=f32), (N_NBR, 1))

    nuc_row = jnp.concatenate(
        [jnp.zeros((N_ELEC * KER,), f32), nuc_embed.reshape(N_ATOMS * KER)]
    ).reshape(1, LANES)

    wh_t = jnp.concatenate(
        [jnp.tile(Wh, (1, 1, N_ELEC)),
         jnp.zeros((LAYERS, EMBED, N_ATOMS * KER), f32)], axis=2)
    bh_t = jnp.concatenate(
        [jnp.tile(bh, (1, N_ELEC)),
         jnp.zeros((LAYERS, N_ATOMS * KER), f32)], axis=1)

    x0 = jnp.concatenate([
        jnp.broadcast_to(elec_embed[0][None], (N_UP, EMBED)),
        jnp.broadcast_to(elec_embed[1][None], (N_DOWN, EMBED)),
    ], axis=0)

    wo_row = Wo.reshape(1, EMBED)
    bo2 = bo.reshape(1, 1)

    grid = (BATCH // B_BLK,)

    def whole(shape):
        nd = len(shape)
        return pl.BlockSpec(shape, lambda i: (0,) * nd)

    out = pl.pallas_call(
        _jastrow_kernel,
        grid=grid,
        in_specs=[
            pl.BlockSpec((B_BLK * N_ELEC, N_NBR * BASIS), lambda i: (i, 0)),
            whole(w_big.shape),
            whole(b_big.shape),
            whole(mask.shape),
            whole(diag.shape),
            whole(sel.shape),
            whole(nuc_row.shape),
            whole(wh_t.shape),
            whole(bh_t.shape),
            whole(Wg.shape),
            whole(bg.shape),
            whole(wo_row.shape),
            whole(x0.shape),
            whole(bo2.shape),
        ],
        out_specs=pl.BlockSpec((B_BLK, 1), lambda i: (i, 0)),
        out_shape=jax.ShapeDtypeStruct((BATCH, 1), f32),
    )(xs_flat, w_big, b_big, mask, diag, sel, nuc_row, wh_t, bh_t, Wg, bg,
      wo_row, x0, bo2)
    return out.reshape(BATCH)
